# Initial kernel scaffold; baseline (speedup 1.0000x reference)
import jax
import jax.numpy as jnp
from jax.experimental import pallas as pl


def kernel(msgs, edge_index, num_nodes):
    raise NotImplementedError("write your pallas kernel here")


# speedup vs baseline: 12.0857x; 12.0857x over previous
import jax
import jax.numpy as jnp
from jax import lax
from jax.experimental import pallas as pl
from jax.experimental.pallas import tpu as pltpu
from jax.experimental.pallas import tpu_sc as plsc

_NUM_NODES = 50000
_N_EDGES = 1600000

_NC = 2
_NS = 16
_NW = _NC * _NS

_LANE = 128
_BPW = 392
_NBLK = _BPW * _NW
_E_PAD = _NBLK * _LANE
_MACRO = 56
_NMAC = _BPW // _MACRO

_NPAD = 50176
_SLICE = _NPAD // _NS


def _make_sc_kernel():
    mesh = plsc.VectorSubcoreMesh(
        core_axis_name="c", subcore_axis_name="s",
        num_cores=_NC, num_subcores=_NS)

    def body(dst_ref, msg_ref, out_ref, idx_v, val_v, buf_v, acc_sh):
        c = lax.axis_index("c")
        s = lax.axis_index("s")
        wid = c * _NS + s

        def _zero(i, carry):
            buf_v[pl.ds(i * 16, 16)] = jnp.zeros((16,), jnp.float32)
            return carry

        lax.fori_loop(0, _SLICE // 16, _zero, None)
        pltpu.sync_copy(buf_v, acc_sh.at[pl.ds(s * _SLICE, _SLICE)])
        plsc.subcore_barrier()

        out_base = c * _NPAD + s * _SLICE

        base = wid * _BPW

        def _macro(m, carry):
            blk = base + m * _MACRO
            pltpu.sync_copy(dst_ref.at[pl.ds(blk, _MACRO)], idx_v)
            pltpu.sync_copy(msg_ref.at[pl.ds(blk, _MACRO)], val_v)

            def _scat(j, c2):
                pltpu.sync_copy(val_v.at[j], acc_sh.at[idx_v.at[j]],
                                add=True)
                return c2

            lax.fori_loop(0, _MACRO, _scat, None)
            return carry

        lax.fori_loop(0, _NMAC, _macro, None)

        plsc.subcore_barrier()

        pltpu.sync_copy(acc_sh.at[pl.ds(s * _SLICE, _SLICE)], buf_v)
        pltpu.sync_copy(buf_v, out_ref.at[pl.ds(out_base, _SLICE)])

    return pl.kernel(
        body,
        out_type=jax.ShapeDtypeStruct((_NC * _NPAD,), jnp.float32),
        mesh=mesh,
        scratch_types=[
            pltpu.VMEM((_MACRO, _LANE), jnp.int32),
            pltpu.VMEM((_MACRO, _LANE), jnp.float32),
            pltpu.VMEM((_SLICE,), jnp.float32),
            pltpu.VMEM_SHARED((_NPAD,), jnp.float32),
        ],
    )


def _combine_body(p_ref, o_ref):
    o_ref[...] = p_ref[0] + p_ref[1]


def kernel(msgs, edge_index, num_nodes):
    del num_nodes
    npad = _E_PAD - _N_EDGES
    pad_idx = jnp.arange(npad, dtype=jnp.int32) % _NUM_NODES
    dst = jnp.concatenate([edge_index[1], pad_idx]).reshape(_NBLK, _LANE)
    vals = jnp.concatenate(
        [msgs, jnp.zeros((npad,), jnp.float32)]).reshape(_NBLK, _LANE)
    partials = _make_sc_kernel()(dst, vals)
    summed = pl.pallas_call(
        _combine_body,
        out_shape=jax.ShapeDtypeStruct((_NPAD // _LANE, _LANE), jnp.float32),
    )(partials.reshape(_NC, _NPAD // _LANE, _LANE))
    return summed.reshape(_NPAD)[:_NUM_NODES]


# speedup vs baseline: 12.1670x; 1.0067x over previous
import jax
import jax.numpy as jnp
from jax import lax
from jax.experimental import pallas as pl
from jax.experimental.pallas import tpu as pltpu
from jax.experimental.pallas import tpu_sc as plsc

_NUM_NODES = 50000
_N_EDGES = 1600000

_NC = 2
_NS = 16
_NW = _NC * _NS

_LANE = 128
_NBLK = _N_EDGES // _LANE
_BPW = _NBLK // _NW
_EXTRA = _NBLK - _BPW * _NW
_MACRO = 26
_NMAC = _BPW // _MACRO
_MEDGE = _MACRO * _LANE

_NPAD = 50176
_SLICE = _NPAD // _NS


def _make_sc_kernel():
    mesh = plsc.VectorSubcoreMesh(
        core_axis_name="c", subcore_axis_name="s",
        num_cores=_NC, num_subcores=_NS)

    def body(dst_ref, msg_ref, out_ref, idx_v, val_v, buf_v, acc_sh):
        c = lax.axis_index("c")
        s = lax.axis_index("s")
        wid = c * _NS + s

        def _zero(i, carry):
            buf_v[pl.ds(i * 16, 16)] = jnp.zeros((16,), jnp.float32)
            return carry

        lax.fori_loop(0, _SLICE // 16, _zero, None)
        pltpu.sync_copy(buf_v, acc_sh.at[pl.ds(s * _SLICE, _SLICE)])
        plsc.subcore_barrier()

        base = (wid * _BPW + jnp.minimum(wid, _EXTRA)) * _LANE

        def _scat(j, c2):
            pltpu.sync_copy(val_v.at[pl.ds(j * _LANE, _LANE)],
                            acc_sh.at[idx_v.at[pl.ds(j * _LANE, _LANE)]],
                            add=True)
            return c2

        def _macro(m, carry):
            e0 = base + m * _MEDGE
            pltpu.sync_copy(dst_ref.at[pl.ds(e0, _MEDGE)], idx_v)
            pltpu.sync_copy(msg_ref.at[pl.ds(e0, _MEDGE)], val_v)
            lax.fori_loop(0, _MACRO, _scat, None)
            return carry

        lax.fori_loop(0, _NMAC, _macro, None)

        @pl.when(wid < _EXTRA)
        def _():
            e0 = base + _BPW * _LANE
            pltpu.sync_copy(dst_ref.at[pl.ds(e0, _LANE)],
                            idx_v.at[pl.ds(0, _LANE)])
            pltpu.sync_copy(msg_ref.at[pl.ds(e0, _LANE)],
                            val_v.at[pl.ds(0, _LANE)])
            pltpu.sync_copy(val_v.at[pl.ds(0, _LANE)],
                            acc_sh.at[idx_v.at[pl.ds(0, _LANE)]],
                            add=True)

        plsc.subcore_barrier()

        pltpu.sync_copy(acc_sh.at[pl.ds(s * _SLICE, _SLICE)], buf_v)
        pltpu.sync_copy(buf_v,
                        out_ref.at[pl.ds(c * _NPAD + s * _SLICE, _SLICE)])

    return pl.kernel(
        body,
        out_type=jax.ShapeDtypeStruct((_NC * _NPAD,), jnp.float32),
        mesh=mesh,
        scratch_types=[
            pltpu.VMEM((_MEDGE,), jnp.int32),
            pltpu.VMEM((_MEDGE,), jnp.float32),
            pltpu.VMEM((_SLICE,), jnp.float32),
            pltpu.VMEM_SHARED((_NPAD,), jnp.float32),
        ],
    )


def _combine_body(p_ref, o_ref):
    o_ref[...] = (p_ref[pl.ds(0, _NUM_NODES)]
                  + p_ref[pl.ds(_NPAD, _NUM_NODES)])


def kernel(msgs, edge_index, num_nodes):
    del num_nodes
    dst = edge_index[1]
    partials = _make_sc_kernel()(dst, msgs)
    return pl.pallas_call(
        _combine_body,
        out_shape=jax.ShapeDtypeStruct((_NUM_NODES,), jnp.float32),
    )(partials)


# speedup vs baseline: 16.7994x; 1.3807x over previous
import jax
import jax.numpy as jnp
from jax import lax
from jax.experimental import pallas as pl
from jax.experimental.pallas import tpu as pltpu
from jax.experimental.pallas import tpu_sc as plsc

_NUM_NODES = 50000
_N_EDGES = 1600000

_NC = 2
_NS = 16
_NW = _NC * _NS

_LANE = 128
_NBLK = _N_EDGES // _LANE
_BPW = _NBLK // _NW
_EXTRA = _NBLK - _BPW * _NW
_WEDGE = _BPW * _LANE

_NPAD = 50176
_SLICE = _NPAD // _NS


def _make_sc_kernel():
    mesh = plsc.VectorSubcoreMesh(
        core_axis_name="c", subcore_axis_name="s",
        num_cores=_NC, num_subcores=_NS)

    def body(dst_ref, msg_ref, out_ref, idx_v, val_v, buf_v, acc_sh):
        c = lax.axis_index("c")
        s = lax.axis_index("s")
        wid = c * _NS + s

        def _zero(i, carry):
            buf_v[pl.ds(i * 16, 16)] = jnp.zeros((16,), jnp.float32)
            return carry

        lax.fori_loop(0, _SLICE // 16, _zero, None)
        pltpu.sync_copy(buf_v, acc_sh.at[pl.ds(s * _SLICE, _SLICE)])
        plsc.subcore_barrier()

        base = (wid * _BPW + jnp.minimum(wid, _EXTRA)) * _LANE
        pltpu.sync_copy(dst_ref.at[pl.ds(base, _WEDGE)], idx_v)
        pltpu.sync_copy(msg_ref.at[pl.ds(base, _WEDGE)], val_v)
        pltpu.sync_copy(val_v, acc_sh.at[idx_v], add=True)

        @pl.when(wid < _EXTRA)
        def _():
            e0 = base + _WEDGE
            pltpu.sync_copy(dst_ref.at[pl.ds(e0, _LANE)],
                            idx_v.at[pl.ds(0, _LANE)])
            pltpu.sync_copy(msg_ref.at[pl.ds(e0, _LANE)],
                            val_v.at[pl.ds(0, _LANE)])
            pltpu.sync_copy(val_v.at[pl.ds(0, _LANE)],
                            acc_sh.at[idx_v.at[pl.ds(0, _LANE)]],
                            add=True)

        plsc.subcore_barrier()

        pltpu.sync_copy(acc_sh.at[pl.ds(s * _SLICE, _SLICE)], buf_v)
        pltpu.sync_copy(buf_v,
                        out_ref.at[pl.ds(c * _NPAD + s * _SLICE, _SLICE)])

    return pl.kernel(
        body,
        out_type=jax.ShapeDtypeStruct((_NC * _NPAD,), jnp.float32),
        mesh=mesh,
        scratch_types=[
            pltpu.VMEM((_WEDGE,), jnp.int32),
            pltpu.VMEM((_WEDGE,), jnp.float32),
            pltpu.VMEM((_SLICE,), jnp.float32),
            pltpu.VMEM_SHARED((_NPAD,), jnp.float32),
        ],
    )


def _combine_body(p_ref, o_ref):
    o_ref[...] = (p_ref[pl.ds(0, _NUM_NODES)]
                  + p_ref[pl.ds(_NPAD, _NUM_NODES)])


def kernel(msgs, edge_index, num_nodes):
    del num_nodes
    dst = edge_index[1]
    partials = _make_sc_kernel()(dst, msgs)
    return pl.pallas_call(
        _combine_body,
        out_shape=jax.ShapeDtypeStruct((_NUM_NODES,), jnp.float32),
    )(partials)


# speedup vs baseline: 38.3071x; 2.2803x over previous
import jax
import jax.numpy as jnp
from jax import lax
from jax.experimental import pallas as pl
from jax.experimental.pallas import tpu as pltpu
from jax.experimental.pallas import tpu_sc as plsc

_NUM_NODES = 50000
_N_EDGES = 1600000

_NC = 2
_NS = 16
_NW = _NC * _NS

_LANE = 128
_NBLK = _N_EDGES // _LANE
_BPW = _NBLK // _NW
_EXTRA = _NBLK - _BPW * _NW
_NWIN = 2
_WEDGE = _BPW * _LANE // _NWIN
_KB = 15

_NPAD = 50176
_SLICE = _NPAD // _NS


def _make_sc_kernel():
    mesh = plsc.VectorSubcoreMesh(
        core_axis_name="c", subcore_axis_name="s",
        num_cores=_NC, num_subcores=_NS)

    def body(ei_ref, msg_ref, out_ref, idx_v, val_v, buf_v, acc_sh, sem):
        c = lax.axis_index("c")
        s = lax.axis_index("s")
        wid = c * _NS + s

        def _zero(i, carry):
            buf_v[pl.ds(i * 16, 16)] = jnp.zeros((16,), jnp.float32)
            return carry

        lax.fori_loop(0, _SLICE // 16, _zero, None)
        pltpu.sync_copy(buf_v, acc_sh.at[pl.ds(s * _SLICE, _SLICE)])
        plsc.subcore_barrier()

        base = (wid * _BPW + jnp.minimum(wid, _EXTRA)) * _LANE

        def _win(m, carry):
            e0 = base + m * _WEDGE
            pltpu.sync_copy(ei_ref.at[:, pl.ds(e0, _WEDGE)], idx_v)
            pltpu.sync_copy(msg_ref.at[pl.ds(e0, _WEDGE)], val_v)

            def _chunk(t, c2):
                k0 = t * _KB
                descs = [
                    pltpu.async_copy(
                        val_v.at[pl.ds((k0 + j) * _LANE, _LANE)],
                        acc_sh.at[idx_v.at[1, pl.ds((k0 + j) * _LANE,
                                                    _LANE)]],
                        sem, add=True)
                    for j in range(_KB)]
                for dsc in descs:
                    dsc.wait()
                return c2

            lax.fori_loop(0, _WEDGE // _LANE // _KB, _chunk, None)
            return carry

        lax.fori_loop(0, _NWIN, _win, None)

        @pl.when(wid < _EXTRA)
        def _():
            e0 = base + _NWIN * _WEDGE
            pltpu.sync_copy(ei_ref.at[:, pl.ds(e0, _LANE)],
                            idx_v.at[:, pl.ds(0, _LANE)])
            pltpu.sync_copy(msg_ref.at[pl.ds(e0, _LANE)],
                            val_v.at[pl.ds(0, _LANE)])
            pltpu.sync_copy(val_v.at[pl.ds(0, _LANE)],
                            acc_sh.at[idx_v.at[1, pl.ds(0, _LANE)]],
                            add=True)

        plsc.subcore_barrier()

        pltpu.sync_copy(acc_sh.at[pl.ds(s * _SLICE, _SLICE)], buf_v)
        pltpu.sync_copy(buf_v,
                        out_ref.at[pl.ds(c * _NPAD + s * _SLICE, _SLICE)])

    return pl.kernel(
        body,
        out_type=jax.ShapeDtypeStruct((_NC * _NPAD,), jnp.float32),
        mesh=mesh,
        scratch_types=[
            pltpu.VMEM((2, _WEDGE), jnp.int32),
            pltpu.VMEM((_WEDGE,), jnp.float32),
            pltpu.VMEM((_SLICE,), jnp.float32),
            pltpu.VMEM_SHARED((_NPAD,), jnp.float32),
            pltpu.SemaphoreType.DMA,
        ],
    )


def _combine_body(p_ref, o_ref):
    o_ref[...] = (p_ref[pl.ds(0, _NUM_NODES)]
                  + p_ref[pl.ds(_NPAD, _NUM_NODES)])


def kernel(msgs, edge_index, num_nodes):
    del num_nodes
    partials = _make_sc_kernel()(edge_index, msgs)
    return pl.pallas_call(
        _combine_body,
        out_shape=jax.ShapeDtypeStruct((_NUM_NODES,), jnp.float32),
    )(partials)


# speedup vs baseline: 47.6215x; 1.2432x over previous
import jax
import jax.numpy as jnp
from jax import lax
from jax.experimental import pallas as pl
from jax.experimental.pallas import tpu as pltpu
from jax.experimental.pallas import tpu_sc as plsc

_NUM_NODES = 50000
_N_EDGES = 1600000

_NC = 2
_NS = 16
_NW = _NC * _NS

_LANE = 128
_NBLK = _N_EDGES // _LANE
_BPW = _NBLK // _NW
_EXTRA = _NBLK - _BPW * _NW
_NWIN = 10
_WEDGE = _BPW * _LANE // _NWIN
_WBLK = _WEDGE // _LANE

_NPAD = 50176
_SLICE = _NPAD // _NS


def _make_sc_kernel():
    mesh = plsc.VectorSubcoreMesh(
        core_axis_name="c", subcore_axis_name="s",
        num_cores=_NC, num_subcores=_NS)

    def body(ei_ref, msg_ref, out_ref, idx_v0, val_v0, idx_v1, val_v1,
             idx_x, val_x, buf_v, acc_sh, semA, semB, semS0, semX):
        c = lax.axis_index("c")
        s = lax.axis_index("s")
        wid = c * _NS + s

        base = (wid * _BPW + jnp.minimum(wid, _EXTRA)) * _LANE
        bufs = ((idx_v0, val_v0, semA), (idx_v1, val_v1, semB))

        def _srcs(m):
            e0 = base + m * _WEDGE
            return (ei_ref.at[:, pl.ds(e0, _WEDGE)],
                    msg_ref.at[pl.ds(e0, _WEDGE)])

        for b in range(2):
            iv, vv, sm = bufs[b]
            ei_src, ms_src = _srcs(b)
            pltpu.async_copy(ei_src, iv, sm)
            pltpu.async_copy(ms_src, vv, sm)

        ex0 = base + _NWIN * _WEDGE

        @pl.when(wid < _EXTRA)
        def _():
            pltpu.async_copy(ei_ref.at[:, pl.ds(ex0, _LANE)], idx_x, semX)
            pltpu.async_copy(msg_ref.at[pl.ds(ex0, _LANE)], val_x, semX)

        def _zero(i, carry):
            buf_v[pl.ds(i * 16, 16)] = jnp.zeros((16,), jnp.float32)
            return carry

        lax.fori_loop(0, _SLICE // 16, _zero, None)
        pltpu.sync_copy(buf_v, acc_sh.at[pl.ds(s * _SLICE, _SLICE)])
        plsc.subcore_barrier()

        @pl.when(wid < _EXTRA)
        def _():
            pltpu.make_async_copy(ei_ref.at[:, pl.ds(ex0, _LANE)],
                                  idx_x, semX).wait()
            pltpu.make_async_copy(msg_ref.at[pl.ds(ex0, _LANE)],
                                  val_x, semX).wait()
            pltpu.async_copy(val_x, acc_sh.at[idx_x.at[1]], semX,
                             add=True)

        def _step(t, carry):
            for b in range(2):
                m = 2 * t + b
                iv, vv, sm = bufs[b]
                ei_src, ms_src = _srcs(m)
                pltpu.make_async_copy(ei_src, iv, sm).wait()
                pltpu.make_async_copy(ms_src, vv, sm).wait()
                descs = [
                    pltpu.async_copy(
                        vv.at[pl.ds(j * _LANE, _LANE)],
                        acc_sh.at[iv.at[1, pl.ds(j * _LANE, _LANE)]],
                        semS0, add=True)
                    for j in range(_WBLK)]
                for dsc in descs:
                    dsc.wait()

                @pl.when(m + 2 < _NWIN)
                def _():
                    ei2, ms2 = _srcs(m + 2)
                    pltpu.async_copy(ei2, iv, sm)
                    pltpu.async_copy(ms2, vv, sm)
            return carry

        lax.fori_loop(0, _NWIN // 2, _step, None)

        @pl.when(wid < _EXTRA)
        def _():
            pltpu.make_async_copy(val_x, acc_sh.at[idx_x.at[1]],
                                  semX).wait()

        plsc.subcore_barrier()

        pltpu.sync_copy(acc_sh.at[pl.ds(s * _SLICE, _SLICE)], buf_v)
        pltpu.sync_copy(buf_v,
                        out_ref.at[pl.ds(c * _NPAD + s * _SLICE, _SLICE)])

    return pl.kernel(
        body,
        out_type=jax.ShapeDtypeStruct((_NC * _NPAD,), jnp.float32),
        mesh=mesh,
        scratch_types=[
            pltpu.VMEM((2, _WEDGE), jnp.int32),
            pltpu.VMEM((_WEDGE,), jnp.float32),
            pltpu.VMEM((2, _WEDGE), jnp.int32),
            pltpu.VMEM((_WEDGE,), jnp.float32),
            pltpu.VMEM((2, _LANE), jnp.int32),
            pltpu.VMEM((_LANE,), jnp.float32),
            pltpu.VMEM((_SLICE,), jnp.float32),
            pltpu.VMEM_SHARED((_NPAD,), jnp.float32),
            pltpu.SemaphoreType.DMA,
            pltpu.SemaphoreType.DMA,
            pltpu.SemaphoreType.DMA,
            pltpu.SemaphoreType.DMA,
        ],
    )


def _combine_body(p_ref, o_ref):
    o_ref[...] = (p_ref[pl.ds(0, _NUM_NODES)]
                  + p_ref[pl.ds(_NPAD, _NUM_NODES)])


def kernel(msgs, edge_index, num_nodes):
    del num_nodes
    partials = _make_sc_kernel()(edge_index, msgs)
    return pl.pallas_call(
        _combine_body,
        out_shape=jax.ShapeDtypeStruct((_NUM_NODES,), jnp.float32),
    )(partials)
